# baseline (speedup 1.0000x reference)
import functools

import jax
import jax.numpy as jnp
from jax.experimental import pallas as pl
from jax.experimental.pallas import tpu as pltpu

_LOG2E = 1.4426950408889634


def _gat_kernel(h_ref, w_ref, wa_ref, adj_ref, out_ref,
                whb_ref, sq_ref, sk_ref, *, alpha, tq, n_tiles):
    i = pl.program_id(0)

    @pl.when(i == 0)
    def _project():
        for c in range(n_tiles):
            hc = h_ref[c * tq:(c + 1) * tq, :]
            wh = jnp.dot(hc, w_ref[...], preferred_element_type=jnp.float32)
            whb_ref[c * tq:(c + 1) * tq, :] = wh.astype(jnp.bfloat16)
            sc = jnp.dot(hc, wa_ref[...], preferred_element_type=jnp.float32)
            sq_ref[c * tq:(c + 1) * tq, :] = sc[:, 0:1]
            sk_ref[0:1, c * tq:(c + 1) * tq] = jnp.transpose(sc[:, 1:2])

    sk = sk_ref[...]
    sq = sq_ref[pl.ds(i * tq, tq), :]
    rm = sq + jnp.max(sk)
    m = jnp.maximum(rm, alpha * rm)

    a1 = (sq - m) * _LOG2E
    a2 = (alpha * sq - m) * _LOG2E
    b1 = sk * _LOG2E
    b2 = sk * (alpha * _LOG2E)
    t = jnp.maximum(a1 + b1, a2 + b2)
    p = jnp.exp2(t) * adj_ref[...]
    denom = jnp.sum(p, axis=1, keepdims=True)

    acc = jnp.dot(p.astype(jnp.bfloat16), whb_ref[...],
                  preferred_element_type=jnp.float32)
    out = acc * pl.reciprocal(denom, approx=False)
    out = jnp.where(out > 0, out, jnp.exp(out) - 1.0)
    out_ref[...] = out


def _row_tile(n, max_tile=512):
    if n <= max_tile:
        return n
    for t in (512, 256, 128):
        if n % t == 0:
            return t
    return n


def kernel(h, W, a, adj):
    alpha = 0.2
    N, f_in = h.shape
    f_out = W.shape[1]
    a_mat = jnp.transpose(a.reshape(2, f_out))
    wa = jnp.dot(W, a_mat)
    hb = h.astype(jnp.bfloat16)
    wb = W.astype(jnp.bfloat16)
    wab = wa.astype(jnp.bfloat16)

    tq = _row_tile(N)
    n_tiles = N // tq

    cost = pl.CostEstimate(
        flops=2 * N * f_in * f_out + 4 * N * f_in + 2 * N * N * f_out
        + 8 * N * N,
        transcendentals=N * N + N * f_out,
        bytes_accessed=4 * N * N + 4 * N * f_out
        + 2 * (N * f_in + f_in * f_out),
    )
    body = functools.partial(_gat_kernel, alpha=alpha, tq=tq, n_tiles=n_tiles)
    out = pl.pallas_call(
        body,
        out_shape=jax.ShapeDtypeStruct((N, f_out), jnp.float32),
        grid=(n_tiles,),
        in_specs=[
            pl.BlockSpec((N, f_in), lambda i: (0, 0)),
            pl.BlockSpec((f_in, f_out), lambda i: (0, 0)),
            pl.BlockSpec((f_in, 2), lambda i: (0, 0)),
            pl.BlockSpec((tq, N), lambda i: (i, 0)),
        ],
        out_specs=pl.BlockSpec((tq, f_out), lambda i: (i, 0)),
        scratch_shapes=[
            pltpu.VMEM((N, f_out), jnp.bfloat16),
            pltpu.VMEM((N, 1), jnp.float32),
            pltpu.VMEM((1, N), jnp.float32),
        ],
        compiler_params=pltpu.CompilerParams(
            dimension_semantics=("arbitrary",)),
        cost_estimate=cost,
    )(hb, wb, wab, adj)
    return out


# speedup vs baseline: 1.1780x; 1.1780x over previous
import functools

import jax
import jax.numpy as jnp
from jax.experimental import pallas as pl
from jax.experimental.pallas import tpu as pltpu

_LOG2E = 1.4426950408889634


def _gat_kernel(h_ref, w_ref, wa_ref, adj_ref, out_ref,
                whb_ref, sq_ref, sk_ref, *, alpha, tq, n_tiles):
    i = pl.program_id(0)

    @pl.when(i == 0)
    def _project():
        for c in range(n_tiles):
            hc = h_ref[c * tq:(c + 1) * tq, :]
            wh = jnp.dot(hc, w_ref[...], preferred_element_type=jnp.float32)
            whb_ref[c * tq:(c + 1) * tq, :] = wh.astype(jnp.bfloat16)
            sc = jnp.dot(hc, wa_ref[...], preferred_element_type=jnp.float32)
            sq_ref[c * tq:(c + 1) * tq, :] = sc[:, 0:1]
            sk_ref[0:1, c * tq:(c + 1) * tq] = jnp.transpose(sc[:, 1:2])

    sk = sk_ref[...]
    sq = sq_ref[pl.ds(i * tq, tq), :]
    rm = sq + jnp.max(sk)
    m = jnp.maximum(rm, alpha * rm)

    a1 = (sq - m) * _LOG2E
    a2 = (alpha * sq - m) * _LOG2E
    b1 = sk * _LOG2E
    b2 = sk * (alpha * _LOG2E)
    t = jnp.maximum(a1 + b1, a2 + b2)
    p = jnp.exp2(t) * adj_ref[...]
    denom = jnp.sum(p, axis=1, keepdims=True)

    acc = jnp.dot(p.astype(jnp.bfloat16), whb_ref[...],
                  preferred_element_type=jnp.float32)
    out = acc * pl.reciprocal(denom, approx=False)
    out = jnp.where(out > 0, out, jnp.exp(out) - 1.0)
    out_ref[...] = out


def _row_tile(n, max_tile=512):
    if n <= max_tile:
        return n
    for t in (512, 256, 128):
        if n % t == 0:
            return t
    return n


def kernel(h, W, a, adj):
    alpha = 0.2
    N, f_in = h.shape
    f_out = W.shape[1]
    a_mat = jnp.transpose(a.reshape(2, f_out))
    wa = jnp.dot(W, a_mat)

    tq = _row_tile(N)
    n_tiles = N // tq

    cost = pl.CostEstimate(
        flops=2 * N * f_in * f_out + 4 * N * f_in + 2 * N * N * f_out
        + 8 * N * N,
        transcendentals=N * N + N * f_out,
        bytes_accessed=4 * (N * N + N * f_in + N * f_out + f_in * f_out),
    )
    body = functools.partial(_gat_kernel, alpha=alpha, tq=tq, n_tiles=n_tiles)
    out = pl.pallas_call(
        body,
        out_shape=jax.ShapeDtypeStruct((N, f_out), jnp.float32),
        grid=(n_tiles,),
        in_specs=[
            pl.BlockSpec((N, f_in), lambda i: (0, 0)),
            pl.BlockSpec((f_in, f_out), lambda i: (0, 0)),
            pl.BlockSpec((f_in, 2), lambda i: (0, 0)),
            pl.BlockSpec((tq, N), lambda i: (i, 0)),
        ],
        out_specs=pl.BlockSpec((tq, f_out), lambda i: (i, 0)),
        scratch_shapes=[
            pltpu.VMEM((N, f_out), jnp.bfloat16),
            pltpu.VMEM((N, 1), jnp.float32),
            pltpu.VMEM((1, N), jnp.float32),
        ],
        compiler_params=pltpu.CompilerParams(
            dimension_semantics=("arbitrary",)),
        cost_estimate=cost,
    )(h, W, wa, adj)
    return out


# speedup vs baseline: 1.1815x; 1.0030x over previous
import functools

import jax
import jax.numpy as jnp
from jax.experimental import pallas as pl
from jax.experimental.pallas import tpu as pltpu

_LOG2E = 1.4426950408889634


def _gat_kernel(h_ref, w_ref, wa_ref, adj_ref, out_ref,
                whb_ref, sq_ref, sk_ref, *, alpha, tq, n_tiles):
    i = pl.program_id(0)

    @pl.when(i == 0)
    def _project():
        for c in range(n_tiles):
            hc = h_ref[c * tq:(c + 1) * tq, :]
            wh = jnp.dot(hc, w_ref[...], preferred_element_type=jnp.float32)
            whb_ref[c * tq:(c + 1) * tq, :] = wh.astype(jnp.bfloat16)
            sc = jnp.dot(hc, wa_ref[...], preferred_element_type=jnp.float32)
            sq_ref[c * tq:(c + 1) * tq, :] = sc[:, 0:1]
            sk_ref[0:1, c * tq:(c + 1) * tq] = jnp.transpose(sc[:, 1:2])

    @pl.when(i > 0)
    def _attend():
        j = i - 1
        sk = sk_ref[...]
        sq = sq_ref[pl.ds(j * tq, tq), :]
        rm = sq + jnp.max(sk)
        m = jnp.maximum(rm, alpha * rm)

        a1 = (sq - m) * _LOG2E
        a2 = (alpha * sq - m) * _LOG2E
        b1 = sk * _LOG2E
        b2 = sk * (alpha * _LOG2E)
        t = jnp.maximum(a1 + b1, a2 + b2)
        p = jnp.exp2(t) * adj_ref[...]
        denom = jnp.sum(p, axis=1, keepdims=True)

        acc = jnp.dot(p.astype(jnp.bfloat16), whb_ref[...],
                      preferred_element_type=jnp.float32)
        out = acc * pl.reciprocal(denom, approx=False)
        out = jnp.where(out > 0, out, jnp.exp(out) - 1.0)
        out_ref[...] = out


def _row_tile(n, max_tile=512):
    if n <= max_tile:
        return n
    for t in (512, 256, 128):
        if n % t == 0:
            return t
    return n


def kernel(h, W, a, adj):
    alpha = 0.2
    N, f_in = h.shape
    f_out = W.shape[1]
    a_mat = jnp.transpose(a.reshape(2, f_out))
    wa = jnp.dot(W, a_mat)

    tq = _row_tile(N)
    n_tiles = N // tq

    cost = pl.CostEstimate(
        flops=2 * N * f_in * f_out + 4 * N * f_in + 2 * N * N * f_out
        + 8 * N * N,
        transcendentals=N * N + N * f_out,
        bytes_accessed=4 * (N * N + N * f_in + N * f_out + f_in * f_out),
    )
    body = functools.partial(_gat_kernel, alpha=alpha, tq=tq, n_tiles=n_tiles)
    out = pl.pallas_call(
        body,
        out_shape=jax.ShapeDtypeStruct((N, f_out), jnp.float32),
        grid=(n_tiles + 1,),
        in_specs=[
            pl.BlockSpec((N, f_in), lambda i: (0, 0)),
            pl.BlockSpec((f_in, f_out), lambda i: (0, 0)),
            pl.BlockSpec((f_in, 2), lambda i: (0, 0)),
            pl.BlockSpec((tq, N), lambda i: (jnp.maximum(i - 1, 0), 0)),
        ],
        out_specs=pl.BlockSpec(
            (tq, f_out), lambda i: (jnp.maximum(i - 1, 0), 0)),
        scratch_shapes=[
            pltpu.VMEM((N, f_out), jnp.bfloat16),
            pltpu.VMEM((N, 1), jnp.float32),
            pltpu.VMEM((1, N), jnp.float32),
        ],
        compiler_params=pltpu.CompilerParams(
            dimension_semantics=("arbitrary",)),
        cost_estimate=cost,
    )(h, W, wa, adj)
    return out


# speedup vs baseline: 1.2640x; 1.0698x over previous
import functools

import jax
import jax.numpy as jnp
from jax.experimental import pallas as pl
from jax.experimental.pallas import tpu as pltpu

_LOG2E = 1.4426950408889634


def _gat_kernel(h_ref, w_ref, a_ref, adj_ref, out_ref,
                whb_ref, sq_ref, sk_ref, *, alpha, tq, n_tiles, f_out):
    i = pl.program_id(0)

    @pl.when(i == 0)
    def _project():
        a_mat = jnp.concatenate(
            [a_ref[0:f_out, :], a_ref[f_out:2 * f_out, :]], axis=1)
        for c in range(n_tiles):
            hc = h_ref[c * tq:(c + 1) * tq, :]
            wh = jnp.dot(hc, w_ref[...], preferred_element_type=jnp.float32)
            whb_ref[c * tq:(c + 1) * tq, :] = wh.astype(jnp.bfloat16)
            sc = jnp.dot(wh, a_mat, preferred_element_type=jnp.float32)
            sq_ref[c * tq:(c + 1) * tq, :] = sc[:, 0:1]
            sk_ref[0:1, c * tq:(c + 1) * tq] = jnp.transpose(sc[:, 1:2])

    @pl.when(i > 0)
    def _attend():
        j = i - 1
        sk = sk_ref[...]
        sq = sq_ref[pl.ds(j * tq, tq), :]
        rm = sq + jnp.max(sk)
        m = jnp.maximum(rm, alpha * rm)

        a1 = (sq - m) * _LOG2E
        a2 = (alpha * sq - m) * _LOG2E
        b1 = sk * _LOG2E
        b2 = sk * (alpha * _LOG2E)
        t = jnp.maximum(a1 + b1, a2 + b2)
        p = jnp.exp2(t) * adj_ref[...]
        denom = jnp.sum(p, axis=1, keepdims=True)

        acc = jnp.dot(p.astype(jnp.bfloat16), whb_ref[...],
                      preferred_element_type=jnp.float32)
        out = acc * pl.reciprocal(denom, approx=False)
        out = jnp.where(out > 0, out, jnp.exp(out) - 1.0)
        out_ref[...] = out


def _row_tile(n, max_tile=512):
    if n <= max_tile:
        return n
    for t in (512, 256, 128):
        if n % t == 0:
            return t
    return n


def kernel(h, W, a, adj):
    alpha = 0.2
    N, f_in = h.shape
    f_out = W.shape[1]

    tq = _row_tile(N)
    n_tiles = N // tq

    cost = pl.CostEstimate(
        flops=2 * N * f_in * f_out + 4 * N * f_out + 2 * N * N * f_out
        + 8 * N * N,
        transcendentals=N * N + N * f_out,
        bytes_accessed=4 * (N * N + N * f_in + N * f_out + f_in * f_out),
    )
    body = functools.partial(_gat_kernel, alpha=alpha, tq=tq,
                             n_tiles=n_tiles, f_out=f_out)
    out = pl.pallas_call(
        body,
        out_shape=jax.ShapeDtypeStruct((N, f_out), jnp.float32),
        grid=(n_tiles + 1,),
        in_specs=[
            pl.BlockSpec((N, f_in), lambda i: (0, 0)),
            pl.BlockSpec((f_in, f_out), lambda i: (0, 0)),
            pl.BlockSpec((2 * f_out, 1), lambda i: (0, 0)),
            pl.BlockSpec((tq, N), lambda i: (jnp.maximum(i - 1, 0), 0)),
        ],
        out_specs=pl.BlockSpec(
            (tq, f_out), lambda i: (jnp.maximum(i - 1, 0), 0)),
        scratch_shapes=[
            pltpu.VMEM((N, f_out), jnp.bfloat16),
            pltpu.VMEM((N, 1), jnp.float32),
            pltpu.VMEM((1, N), jnp.float32),
        ],
        compiler_params=pltpu.CompilerParams(
            dimension_semantics=("arbitrary",)),
        cost_estimate=cost,
    )(h, W, a, adj)
    return out
